# Initial kernel scaffold; baseline (speedup 1.0000x reference)
import jax
import jax.numpy as jnp
from jax.experimental import pallas as pl


def kernel(X, y, table):
    raise NotImplementedError("write your pallas kernel here")


# speedup vs baseline: 1.7053x; 1.7053x over previous
import jax
import jax.numpy as jnp
from jax import lax
from jax.experimental import pallas as pl
from jax.experimental.pallas import tpu as pltpu, tpu_sc as plsc

V = 1000
N = 1024 * 50
NC = 2
NS = 16
NW = NC * NS
PW = N // NW
R = 32
NCHUNK = PW // R
G = 16


def _lse_body(table_ref, lse_ref):
    x = table_ref[...]
    m = jnp.max(x, axis=1, keepdims=True)
    s = jnp.sum(jnp.exp(x - m), axis=1, keepdims=True)
    lse_ref[...] = m + jnp.log(s)


def _row_lse(table):
    return pl.pallas_call(
        _lse_body,
        out_shape=jax.ShapeDtypeStruct((V, 1), jnp.float32),
    )(table)


def _sc_body(table_h, x_h, y_h, lse_h, out_h, part_h,
             idx_v, y_v, lse_v, buf0, buf1, acc_v,
             semg0, semg1, sems0, sems1):
    wid = lax.axis_index("s") * NC + lax.axis_index("c")
    base = wid * PW

    pltpu.sync_copy(x_h.at[pl.ds(base, PW)], idx_v)
    pltpu.sync_copy(y_h.at[pl.ds(base, PW)], y_v)
    pltpu.sync_copy(lse_h, lse_v)

    bufs = (buf0, buf1)
    semg = (semg0, semg1)
    sems = (sems0, sems1)

    def g_copy(c, b):
        return pltpu.make_async_copy(
            table_h.at[idx_v.at[pl.ds(c * R, R)]], bufs[b], semg[b])

    def s_copy(c, b):
        return pltpu.make_async_copy(
            bufs[b], out_h.at[pl.ds(base + c * R, R)], sems[b])

    lanes = lax.iota(jnp.int32, G)

    def chunk_acc(c, b, acc):
        for t0 in (0, G):
            x16 = idx_v[pl.ds(c * R + t0, G)]
            y16 = y_v[pl.ds(c * R + t0, G)]
            row = plsc.load_gather(bufs[b], [lanes + t0, y16])
            ls = plsc.load_gather(lse_v, [x16])
            acc = acc + (ls - row)
        return acc

    for b in range(2):
        g_copy(b, b).start()

    def cbody(c0, acc):
        for b in range(2):
            c = c0 * 2 + b
            g_copy(c, b).wait()
            acc = chunk_acc(c, b, acc)
            s_copy(c, b).start()
            s_copy(c, b).wait()
            g_copy(c + 2, b).start()
        return acc

    acc = lax.fori_loop(0, NCHUNK // 2 - 1, cbody,
                        jnp.zeros((G,), jnp.float32))

    for b in range(2):
        c = NCHUNK - 2 + b
        g_copy(c, b).wait()
        acc = chunk_acc(c, b, acc)
        s_copy(c, b).start()

    acc_v[...] = acc
    for b in range(2):
        s_copy(NCHUNK - 2 + b, b).wait()
    pltpu.sync_copy(acc_v, part_h.at[wid])


def _sc_gather(table, xf, yf, lse):
    mesh = plsc.VectorSubcoreMesh(
        core_axis_name="c", subcore_axis_name="s",
        num_cores=NC, num_subcores=NS)
    f = pl.kernel(
        _sc_body,
        out_type=(
            jax.ShapeDtypeStruct((N, V), jnp.float32),
            jax.ShapeDtypeStruct((NW, G), jnp.float32),
        ),
        mesh=mesh,
        compiler_params=pltpu.CompilerParams(
            use_tc_tiling_on_sc=False, needs_layout_passes=False),
        scratch_types=[
            pltpu.VMEM((PW,), jnp.int32),
            pltpu.VMEM((PW,), jnp.int32),
            pltpu.VMEM((V,), jnp.float32),
            pltpu.VMEM((R, V), jnp.float32),
            pltpu.VMEM((R, V), jnp.float32),
            pltpu.VMEM((G,), jnp.float32),
            pltpu.SemaphoreType.DMA,
            pltpu.SemaphoreType.DMA,
            pltpu.SemaphoreType.DMA,
            pltpu.SemaphoreType.DMA,
        ],
    )
    return f(table, xf, yf, lse)


def _sum_body(p_ref, o_ref):
    o_ref[...] = jnp.sum(p_ref[...], axis=(0, 1), keepdims=True) * (1.0 / N)


def _final_loss(part):
    return pl.pallas_call(
        _sum_body,
        out_shape=jax.ShapeDtypeStruct((1, 1), jnp.float32),
    )(part)[0, 0]


def kernel(X, y, table):
    xf = X.reshape(-1).astype(jnp.int32)
    yf = y.reshape(-1).astype(jnp.int32)
    lse = _row_lse(table).reshape(-1)
    logits, part = _sc_gather(table, xf, yf, lse)
    loss = _final_loss(part)
    return logits, loss


# speedup vs baseline: 2.4854x; 1.4575x over previous
import jax
import jax.numpy as jnp
from jax import lax
from jax.experimental import pallas as pl
from jax.experimental.pallas import tpu as pltpu, tpu_sc as plsc

V = 1000
VP = 1024
N = 1024 * 50
NC = 2
NS = 16
NW = NC * NS
PW = N // NW
R = 16
NCHUNK = PW // R
G = 16


def _lse_body(table_ref, lse_ref):
    x = table_ref[...]
    m = jnp.max(x, axis=1, keepdims=True)
    s = jnp.sum(jnp.exp(x - m), axis=1, keepdims=True)
    lse_ref[...] = m + jnp.log(s)


def _row_lse(table):
    return pl.pallas_call(
        _lse_body,
        out_shape=jax.ShapeDtypeStruct((V, 1), jnp.float32),
    )(table)


def _sc_body(tpad_h, x_h, y_h, lse_h, out_h, part_h,
             idx_v, y_v, lse_v, pad0, pad1, cb0, cb1, acc_v,
             semg0, semg1, sems0, sems1):
    wid = lax.axis_index("s") * NC + lax.axis_index("c")
    base = wid * PW

    pltpu.sync_copy(x_h.at[pl.ds(base, PW)], idx_v)
    pltpu.sync_copy(y_h.at[pl.ds(base, PW)], y_v)
    pltpu.sync_copy(lse_h, lse_v)

    pads = (pad0, pad1)
    cbs = (cb0, cb1)
    semg = (semg0, semg1)
    sems = (sems0, sems1)

    def g_copy(c, b):
        return pltpu.make_async_copy(
            tpad_h.at[idx_v.at[pl.ds(c * R, R)]], pads[b], semg[b])

    def s_copy(c, b):
        return pltpu.make_async_copy(
            cbs[b], out_h.at[pl.ds(base + c * R, R)], sems[b])

    lanes = lax.iota(jnp.int32, G)

    def compact(b):
        def rbody(r, _):
            for j in range(62):
                cbs[b][r, pl.ds(j * G, G)] = pads[b][r, pl.ds(j * G, G)]
            cbs[b][r, pl.ds(984, G)] = pads[b][r, pl.ds(984, G)]
            return 0
        lax.fori_loop(0, R, rbody, 0)

    def chunk_acc(c, b, acc):
        x16 = idx_v[pl.ds(c * R, G)]
        y16 = y_v[pl.ds(c * R, G)]
        row = plsc.load_gather(pads[b], [lanes, y16])
        ls = plsc.load_gather(lse_v, [x16])
        return acc + (ls - row)

    for b in range(2):
        g_copy(b, b).start()

    acc = jnp.zeros((G,), jnp.float32)
    for c in range(2):
        b = c
        g_copy(c, b).wait()
        compact(b)
        acc = chunk_acc(c, b, acc)
        s_copy(c, b).start()
        g_copy(c + 2, b).start()

    def cbody(c0, acc):
        for b2 in range(2):
            c = 2 + c0 * 2 + b2
            b = (2 + b2) % 2
            g_copy(c, b).wait()
            s_copy(c - 2, b).wait()
            compact(b)
            acc = chunk_acc(c, b, acc)
            s_copy(c, b).start()

            @pl.when(c + 2 < NCHUNK)
            def _():
                g_copy(c + 2, b).start()
        return acc

    acc = lax.fori_loop(0, (NCHUNK - 2) // 2, cbody, acc)

    acc_v[...] = acc
    for b in range(2):
        s_copy(NCHUNK - 2 + b, b).wait()
    pltpu.sync_copy(acc_v, part_h.at[pl.ds(wid * G, G)])


def _sc_gather(tpad, xf, yf, lse):
    mesh = plsc.VectorSubcoreMesh(
        core_axis_name="c", subcore_axis_name="s",
        num_cores=NC, num_subcores=NS)
    f = pl.kernel(
        _sc_body,
        out_type=(
            jax.ShapeDtypeStruct((N, V), jnp.float32),
            jax.ShapeDtypeStruct((NW * G,), jnp.float32),
        ),
        mesh=mesh,
        compiler_params=pltpu.CompilerParams(needs_layout_passes=False),
        scratch_types=[
            pltpu.VMEM((PW,), jnp.int32),
            pltpu.VMEM((PW,), jnp.int32),
            pltpu.VMEM((V,), jnp.float32),
            pltpu.VMEM((R, VP), jnp.float32),
            pltpu.VMEM((R, VP), jnp.float32),
            pltpu.VMEM((R, V), jnp.float32),
            pltpu.VMEM((R, V), jnp.float32),
            pltpu.VMEM((G,), jnp.float32),
            pltpu.SemaphoreType.DMA,
            pltpu.SemaphoreType.DMA,
            pltpu.SemaphoreType.DMA,
            pltpu.SemaphoreType.DMA,
        ],
    )
    return f(tpad, xf, yf, lse)


def _sum_body(p_ref, o_ref):
    o_ref[...] = jnp.sum(p_ref[...], axis=(0, 1), keepdims=True) * (1.0 / N)


def _final_loss(part):
    return pl.pallas_call(
        _sum_body,
        out_shape=jax.ShapeDtypeStruct((1, 1), jnp.float32),
    )(part)[0, 0]


def kernel(X, y, table):
    xf = X.reshape(-1).astype(jnp.int32)
    yf = y.reshape(-1).astype(jnp.int32)
    lse = _row_lse(table).reshape(-1)
    tpad = jnp.pad(table, ((0, 0), (0, VP - V)))
    logits, part = _sc_gather(tpad, xf, yf, lse)
    loss = _final_loss(part.reshape(NW, G))
    return logits, loss
